# baseline (speedup 1.0000x reference)
import jax
import jax.numpy as jnp
from jax.experimental import pallas as pl
from jax.experimental.pallas import tpu as pltpu

_NUM_C2 = 1000
_DIM_V = 64
_BB = 64


def _um_kernel(c2x, dx, rx, D_t, v_d, v_c2, R_t, W_ih, W_hh, b_ih, b_hh,
               W1a, b1a, W1b, b1b, W2a, b2a, W2b, b2b,
               alpha_o, h_o, c2_o, h_st, c2_st):
    i = pl.program_id(0)
    s = pl.program_id(1)

    @pl.when(s == 0)
    def _():
        h_st[...] = jnp.zeros_like(h_st)
        c2_st[...] = jnp.zeros_like(c2_st)

    d_t = dx[0, :, 0:1]
    r_t = rx[0, :, 0:1]
    c2_t = c2x[0, :, 0:1]

    iota = jax.lax.broadcasted_iota(jnp.int32, (_BB, _NUM_C2), 1)

    gamma = jnp.sum(jnp.where(iota == d_t, D_t[...], 0.0),
                    axis=1, keepdims=True)
    vd = gamma * v_d[...]
    vr = jnp.where(r_t == 1, R_t[1:2, :], R_t[0:1, :])

    def dot_t(a, b):
        return jax.lax.dot_general(a, b, (((1,), (1,)), ((), ())),
                                   preferred_element_type=jnp.float32)

    h = h_st[...]
    x = jnp.concatenate([vd, vr], axis=1)
    gi = dot_t(x, W_ih[...]) + b_ih[...]
    gh = dot_t(h, W_hh[...]) + b_hh[...]
    r_g = jax.nn.sigmoid(gi[:, 0:64] + gh[:, 0:64])
    z_g = jax.nn.sigmoid(gi[:, 64:128] + gh[:, 64:128])
    n_g = jnp.tanh(gi[:, 128:192] + r_g * gh[:, 128:192])
    h_new = (1.0 - z_g) * n_g + z_g * h
    h_st[...] = h_new
    h_o[:, s, 0, :] = h_new

    t1 = jax.nn.relu(dot_t(h_new, W1a[...]) + b1a[...])
    alpha_o[:, s, 0, :] = dot_t(t1, W1b[...])[:, 0:1] + b1b[0, 0]

    ohc2 = iota == c2_t
    c2s = c2_st[...]
    beta2 = jnp.sum(jnp.where(ohc2, c2s, 0.0), axis=1, keepdims=True)
    zcat = jnp.concatenate([beta2 * v_c2[...], vd, vr], axis=1)
    pre = jax.nn.relu(dot_t(zcat, W2a[...]) + b2a[...])
    newv = dot_t(pre, W2b[...])[:, 0:1] + b2b[0, 0]
    c2n = jnp.where(ohc2, newv, c2s)
    c2_st[...] = c2n
    c2_o[:, s, 0, :] = c2n


def kernel(c1_seq, c2_seq, c4_seq, d_seq, r_seq, D_table, v_d, v_c2, R_table,
           W_ih, W_hh, b_ih, b_hh, W1a, b1a, W1b, b1b, W2a, b2a, W2b, b2b):
    del c1_seq, c4_seq
    B, S = c2_seq.shape
    NB = B // _BB
    f32 = jnp.float32

    c2x = jnp.transpose(c2_seq, (1, 0)).reshape(S, B, 1).astype(jnp.int32)
    dx = jnp.transpose(d_seq, (1, 0)).reshape(S, B, 1).astype(jnp.int32)
    rx = jnp.transpose(r_seq, (1, 0)).reshape(S, B, 1).astype(jnp.int32)

    pad7 = jnp.zeros((7, _DIM_V), f32)
    args = (c2x, dx, rx, D_table.reshape(1, -1), v_d.reshape(1, -1),
            v_c2.reshape(1, -1),
            R_table, W_ih, W_hh, b_ih.reshape(1, -1), b_hh.reshape(1, -1),
            W1a, b1a.reshape(1, -1), jnp.concatenate([W1b, pad7], axis=0),
            b1b.reshape(1, -1),
            W2a, b2a.reshape(1, -1), jnp.concatenate([W2b, pad7], axis=0),
            b2b.reshape(1, -1))

    def full(a):
        n = a.ndim
        return pl.BlockSpec(a.shape, lambda i, s, n=n: (0,) * n)

    idx_spec = pl.BlockSpec((1, _BB, 1), lambda i, s: (s, i, 0))
    in_specs = [idx_spec, idx_spec, idx_spec] + [full(a) for a in args[3:]]

    out_shape = (
        jax.ShapeDtypeStruct((B, S, 1, 1), f32),
        jax.ShapeDtypeStruct((B, S, 1, _DIM_V), f32),
        jax.ShapeDtypeStruct((B, S, 1, _NUM_C2), f32),
    )
    out_specs = (
        pl.BlockSpec((_BB, S, 1, 1), lambda i, s: (i, 0, 0, 0)),
        pl.BlockSpec((_BB, S, 1, _DIM_V), lambda i, s: (i, 0, 0, 0)),
        pl.BlockSpec((_BB, S, 1, _NUM_C2), lambda i, s: (i, 0, 0, 0)),
    )
    alpha4, h4, c24 = pl.pallas_call(
        _um_kernel,
        grid=(NB, S),
        in_specs=in_specs,
        out_specs=out_specs,
        out_shape=out_shape,
        scratch_shapes=[pltpu.VMEM((_BB, _DIM_V), f32),
                        pltpu.VMEM((_BB, _NUM_C2), f32)],
        compiler_params=pltpu.CompilerParams(
            dimension_semantics=("parallel", "arbitrary")),
    )(*args)
    return (alpha4.reshape(B, S), h4.reshape(B, S, _DIM_V),
            c24.reshape(B, S, _NUM_C2))


# speedup vs baseline: 1.1580x; 1.1580x over previous
import jax
import jax.numpy as jnp
from jax.experimental import pallas as pl
from jax.experimental.pallas import tpu as pltpu

_NUM_C2 = 1000
_DIM_V = 64
_BB = 128


def _um_kernel(c2x, dx, rx, D_t, v_d, v_c2, R_t, W_ih, W_hh, b_ih, b_hh,
               W1a, b1a, W1b, b1b, W2a, b2a, W2b, b2b,
               alpha_o, h_o, c2_o, h_st, c2_st):
    i = pl.program_id(0)
    s = pl.program_id(1)

    @pl.when(s == 0)
    def _():
        h_st[...] = jnp.zeros_like(h_st)
        c2_st[...] = jnp.zeros_like(c2_st)

    d_t = dx[0, :, 0:1]
    r_t = rx[0, :, 0:1]
    c2_t = c2x[0, :, 0:1]

    iota = jax.lax.broadcasted_iota(jnp.int32, (_BB, _NUM_C2), 1)

    gamma = jnp.sum(jnp.where(iota == d_t, D_t[...], 0.0),
                    axis=1, keepdims=True)
    vd = gamma * v_d[...]
    vr = jnp.where(r_t == 1, R_t[1:2, :], R_t[0:1, :])

    def dot_t(a, b):
        return jax.lax.dot_general(a, b, (((1,), (1,)), ((), ())),
                                   preferred_element_type=jnp.float32)

    h = h_st[...]
    x = jnp.concatenate([vd, vr], axis=1)
    gi = dot_t(x, W_ih[...]) + b_ih[...]
    gh = dot_t(h, W_hh[...]) + b_hh[...]
    r_g = jax.nn.sigmoid(gi[:, 0:64] + gh[:, 0:64])
    z_g = jax.nn.sigmoid(gi[:, 64:128] + gh[:, 64:128])
    n_g = jnp.tanh(gi[:, 128:192] + r_g * gh[:, 128:192])
    h_new = (1.0 - z_g) * n_g + z_g * h
    h_st[...] = h_new
    h_o[:, 0, 0, :] = h_new

    t1 = jax.nn.relu(dot_t(h_new, W1a[...]) + b1a[...])
    alpha_o[:, 0, 0, :] = dot_t(t1, W1b[...])[:, 0:1] + b1b[0, 0]

    ohc2 = iota == c2_t
    c2s = c2_st[...]
    beta2 = jnp.sum(jnp.where(ohc2, c2s, 0.0), axis=1, keepdims=True)
    zcat = jnp.concatenate([beta2 * v_c2[...], vd, vr], axis=1)
    pre = jax.nn.relu(dot_t(zcat, W2a[...]) + b2a[...])
    newv = dot_t(pre, W2b[...])[:, 0:1] + b2b[0, 0]
    c2n = jnp.where(ohc2, newv, c2s)
    c2_st[...] = c2n
    c2_o[:, s, 0, :] = c2n


def kernel(c1_seq, c2_seq, c4_seq, d_seq, r_seq, D_table, v_d, v_c2, R_table,
           W_ih, W_hh, b_ih, b_hh, W1a, b1a, W1b, b1b, W2a, b2a, W2b, b2b):
    del c1_seq, c4_seq
    B, S = c2_seq.shape
    NB = B // _BB
    f32 = jnp.float32

    c2x = jnp.transpose(c2_seq, (1, 0)).reshape(S, B, 1).astype(jnp.int32)
    dx = jnp.transpose(d_seq, (1, 0)).reshape(S, B, 1).astype(jnp.int32)
    rx = jnp.transpose(r_seq, (1, 0)).reshape(S, B, 1).astype(jnp.int32)

    pad7 = jnp.zeros((7, _DIM_V), f32)
    args = (c2x, dx, rx, D_table.reshape(1, -1), v_d.reshape(1, -1),
            v_c2.reshape(1, -1),
            R_table, W_ih, W_hh, b_ih.reshape(1, -1), b_hh.reshape(1, -1),
            W1a, b1a.reshape(1, -1), jnp.concatenate([W1b, pad7], axis=0),
            b1b.reshape(1, -1),
            W2a, b2a.reshape(1, -1), jnp.concatenate([W2b, pad7], axis=0),
            b2b.reshape(1, -1))

    def full(a):
        n = a.ndim
        return pl.BlockSpec(a.shape, lambda i, s, n=n: (0,) * n)

    idx_spec = pl.BlockSpec((1, _BB, 1), lambda i, s: (s, i, 0))
    in_specs = [idx_spec, idx_spec, idx_spec] + [full(a) for a in args[3:]]

    out_shape = (
        jax.ShapeDtypeStruct((B, S, 1, 1), f32),
        jax.ShapeDtypeStruct((B, S, 1, _DIM_V), f32),
        jax.ShapeDtypeStruct((B, S, 1, _NUM_C2), f32),
    )
    out_specs = (
        pl.BlockSpec((_BB, 1, 1, 1), lambda i, s: (i, s, 0, 0)),
        pl.BlockSpec((_BB, 1, 1, _DIM_V), lambda i, s: (i, s, 0, 0)),
        pl.BlockSpec((_BB, S, 1, _NUM_C2), lambda i, s: (i, 0, 0, 0)),
    )
    alpha4, h4, c24 = pl.pallas_call(
        _um_kernel,
        grid=(NB, S),
        in_specs=in_specs,
        out_specs=out_specs,
        out_shape=out_shape,
        scratch_shapes=[pltpu.VMEM((_BB, _DIM_V), f32),
                        pltpu.VMEM((_BB, _NUM_C2), f32)],
        compiler_params=pltpu.CompilerParams(
            dimension_semantics=("parallel", "arbitrary"),
            vmem_limit_bytes=128 * 1024 * 1024),
    )(*args)
    return (alpha4.reshape(B, S), h4.reshape(B, S, _DIM_V),
            c24.reshape(B, S, _NUM_C2))


# speedup vs baseline: 1.5090x; 1.3032x over previous
import jax
import jax.numpy as jnp
from jax.experimental import pallas as pl
from jax.experimental.pallas import tpu as pltpu

_NUM_C2 = 1000
_DIM_V = 64
_BB = 256
_TC = 10


def _um_kernel(c2b, db, rb, D_t, v_d, v_c2, R_t, W_ih, W_hh, b_ih, b_hh,
               W1a, b1a, W1b, b1b, W2a, b2a, W2b, b2b,
               alpha_o, h_o, c2_o, h_st, c2_st, hist_st):
    sc = pl.program_id(1)

    @pl.when(sc == 0)
    def _():
        h_st[...] = jnp.zeros_like(h_st)
        c2_st[...] = jnp.zeros_like(c2_st)
        hist_st[...] = jnp.zeros_like(hist_st)

    c2all = c2b[...]
    dall = db[...]
    rall = rb[...]
    S = c2all.shape[1]

    iota_s = jax.lax.broadcasted_iota(jnp.int32, c2all.shape, 1)
    iota_c = jax.lax.broadcasted_iota(jnp.int32, (_BB, _NUM_C2), 1)

    def dot_t(a, b):
        return jax.lax.dot_general(a, b, (((1,), (1,)), ((), ())),
                                   preferred_element_type=jnp.float32)

    def step(tloc, _):
        t = sc * _TC + tloc
        sel_t = iota_s == t
        c2_t = jnp.sum(jnp.where(sel_t, c2all, 0), axis=1, keepdims=True)
        d_t = jnp.sum(jnp.where(sel_t, dall, 0), axis=1, keepdims=True)
        r_t = jnp.sum(jnp.where(sel_t, rall, 0), axis=1, keepdims=True)

        gamma = jnp.sum(jnp.where(iota_c == d_t, D_t[...], 0.0),
                        axis=1, keepdims=True)
        vd = gamma * v_d[...]
        vr = jnp.where(r_t == 1, R_t[1:2, :], R_t[0:1, :])

        h = h_st[...]
        x = jnp.concatenate([vd, vr], axis=1)
        gi = dot_t(x, W_ih[...]) + b_ih[...]
        gh = dot_t(h, W_hh[...]) + b_hh[...]
        r_g = jax.nn.sigmoid(gi[:, 0:64] + gh[:, 0:64])
        z_g = jax.nn.sigmoid(gi[:, 64:128] + gh[:, 64:128])
        n_g = jnp.tanh(gi[:, 128:192] + r_g * gh[:, 128:192])
        h_new = (1.0 - z_g) * n_g + z_g * h
        h_st[...] = h_new
        h_o[:, tloc, 0, :] = h_new

        t1 = jax.nn.relu(dot_t(h_new, W1a[...]) + b1a[...])
        alpha_o[:, tloc, 0, :] = dot_t(t1, W1b[...])[:, 0:1] + b1b[0, 0]

        hist = hist_st[...]
        match = jnp.logical_and(c2all == c2_t, iota_s < t)
        last = jnp.max(jnp.where(match, iota_s, -1), axis=1, keepdims=True)
        beta2 = jnp.sum(jnp.where(iota_s == last, hist, 0.0),
                        axis=1, keepdims=True)

        zcat = jnp.concatenate([beta2 * v_c2[...], vd, vr], axis=1)
        pre = jax.nn.relu(dot_t(zcat, W2a[...]) + b2a[...])
        newv = dot_t(pre, W2b[...])[:, 0:1] + b2b[0, 0]

        hist_st[...] = jnp.where(sel_t, newv, hist)
        c2n = jnp.where(iota_c == c2_t, newv, c2_st[...])
        c2_st[...] = c2n
        c2_o[:, tloc, 0, :] = c2n
        return 0

    jax.lax.fori_loop(0, _TC, step, 0)


def kernel(c1_seq, c2_seq, c4_seq, d_seq, r_seq, D_table, v_d, v_c2, R_table,
           W_ih, W_hh, b_ih, b_hh, W1a, b1a, W1b, b1b, W2a, b2a, W2b, b2b):
    del c1_seq, c4_seq
    B, S = c2_seq.shape
    NB = B // _BB
    f32 = jnp.float32

    pad7 = jnp.zeros((7, _DIM_V), f32)
    args = (c2_seq.astype(jnp.int32), d_seq.astype(jnp.int32),
            r_seq.astype(jnp.int32),
            D_table.reshape(1, -1), v_d.reshape(1, -1), v_c2.reshape(1, -1),
            R_table, W_ih, W_hh, b_ih.reshape(1, -1), b_hh.reshape(1, -1),
            W1a, b1a.reshape(1, -1), jnp.concatenate([W1b, pad7], axis=0),
            b1b.reshape(1, -1),
            W2a, b2a.reshape(1, -1), jnp.concatenate([W2b, pad7], axis=0),
            b2b.reshape(1, -1))

    def full(a):
        n = a.ndim
        return pl.BlockSpec(a.shape, lambda i, sc, n=n: (0,) * n)

    idx_spec = pl.BlockSpec((_BB, S), lambda i, sc: (i, 0))
    in_specs = [idx_spec, idx_spec, idx_spec] + [full(a) for a in args[3:]]

    out_shape = (
        jax.ShapeDtypeStruct((B, S, 1, 1), f32),
        jax.ShapeDtypeStruct((B, S, 1, _DIM_V), f32),
        jax.ShapeDtypeStruct((B, S, 1, _NUM_C2), f32),
    )
    out_specs = (
        pl.BlockSpec((_BB, _TC, 1, 1), lambda i, sc: (i, sc, 0, 0)),
        pl.BlockSpec((_BB, _TC, 1, _DIM_V), lambda i, sc: (i, sc, 0, 0)),
        pl.BlockSpec((_BB, _TC, 1, _NUM_C2), lambda i, sc: (i, sc, 0, 0)),
    )
    alpha4, h4, c24 = pl.pallas_call(
        _um_kernel,
        grid=(NB, S // _TC),
        in_specs=in_specs,
        out_specs=out_specs,
        out_shape=out_shape,
        scratch_shapes=[pltpu.VMEM((_BB, _DIM_V), f32),
                        pltpu.VMEM((_BB, _NUM_C2), f32),
                        pltpu.VMEM((_BB, S), f32)],
        compiler_params=pltpu.CompilerParams(
            dimension_semantics=("parallel", "arbitrary"),
            vmem_limit_bytes=128 * 1024 * 1024),
    )(*args)
    return (alpha4.reshape(B, S), h4.reshape(B, S, _DIM_V),
            c24.reshape(B, S, _NUM_C2))


# speedup vs baseline: 1.5935x; 1.0560x over previous
import jax
import jax.numpy as jnp
from jax.experimental import pallas as pl
from jax.experimental.pallas import tpu as pltpu

_NUM_C2 = 1000
_DIM_V = 64
_BB = 512
_TC = 10


def _um_kernel(c2b, db, rb, D_t, v_d, v_c2, R_t, W_ih, W_hh, b_ih, b_hh,
               W1a, b1a, W1b, b1b, W2a, b2a, W2b, b2b,
               alpha_o, h_o, c2_o, h_st, c2_st, hist_st):
    sc = pl.program_id(1)

    @pl.when(sc == 0)
    def _():
        h_st[...] = jnp.zeros_like(h_st)
        c2_st[...] = jnp.zeros_like(c2_st)
        hist_st[...] = jnp.zeros_like(hist_st)

    c2all = c2b[...]
    dall = db[...]
    rall = rb[...]
    S = c2all.shape[1]

    iota_s = jax.lax.broadcasted_iota(jnp.int32, c2all.shape, 1)
    iota_c = jax.lax.broadcasted_iota(jnp.int32, (_BB, _NUM_C2), 1)

    def dot_t(a, b):
        return jax.lax.dot_general(a, b, (((1,), (1,)), ((), ())),
                                   preferred_element_type=jnp.float32)

    def step(tloc, _):
        t = sc * _TC + tloc
        sel_t = iota_s == t
        c2_t = jnp.sum(jnp.where(sel_t, c2all, 0), axis=1, keepdims=True)
        d_t = jnp.sum(jnp.where(sel_t, dall, 0), axis=1, keepdims=True)
        r_t = jnp.sum(jnp.where(sel_t, rall, 0), axis=1, keepdims=True)

        gamma = jnp.sum(jnp.where(iota_c == d_t, D_t[...], 0.0),
                        axis=1, keepdims=True)
        vd = gamma * v_d[...]
        vr = jnp.where(r_t == 1, R_t[1:2, :], R_t[0:1, :])

        h = h_st[...]
        x = jnp.concatenate([vd, vr], axis=1)
        gi = dot_t(x, W_ih[...]) + b_ih[...]
        gh = dot_t(h, W_hh[...]) + b_hh[...]
        r_g = jax.nn.sigmoid(gi[:, 0:64] + gh[:, 0:64])
        z_g = jax.nn.sigmoid(gi[:, 64:128] + gh[:, 64:128])
        n_g = jnp.tanh(gi[:, 128:192] + r_g * gh[:, 128:192])
        h_new = (1.0 - z_g) * n_g + z_g * h
        h_st[...] = h_new
        h_o[:, tloc, 0, :] = h_new

        t1 = jax.nn.relu(dot_t(h_new, W1a[...]) + b1a[...])
        alpha_o[:, tloc, 0, :] = dot_t(t1, W1b[...])[:, 0:1] + b1b[0, 0]

        hist = hist_st[...]
        match = jnp.logical_and(c2all == c2_t, iota_s < t)
        last = jnp.max(jnp.where(match, iota_s, -1), axis=1, keepdims=True)
        beta2 = jnp.sum(jnp.where(iota_s == last, hist, 0.0),
                        axis=1, keepdims=True)

        zcat = jnp.concatenate([beta2 * v_c2[...], vd, vr], axis=1)
        pre = jax.nn.relu(dot_t(zcat, W2a[...]) + b2a[...])
        newv = dot_t(pre, W2b[...])[:, 0:1] + b2b[0, 0]

        hist_st[...] = jnp.where(sel_t, newv, hist)
        c2n = jnp.where(iota_c == c2_t, newv, c2_st[...])
        c2_st[...] = c2n
        c2_o[:, tloc, 0, :] = c2n
        return 0

    jax.lax.fori_loop(0, _TC, step, 0)


def kernel(c1_seq, c2_seq, c4_seq, d_seq, r_seq, D_table, v_d, v_c2, R_table,
           W_ih, W_hh, b_ih, b_hh, W1a, b1a, W1b, b1b, W2a, b2a, W2b, b2b):
    del c1_seq, c4_seq
    B, S = c2_seq.shape
    NB = B // _BB
    f32 = jnp.float32

    pad7 = jnp.zeros((7, _DIM_V), f32)
    args = (c2_seq.astype(jnp.int32), d_seq.astype(jnp.int32),
            r_seq.astype(jnp.int32),
            D_table.reshape(1, -1), v_d.reshape(1, -1), v_c2.reshape(1, -1),
            R_table, W_ih, W_hh, b_ih.reshape(1, -1), b_hh.reshape(1, -1),
            W1a, b1a.reshape(1, -1), jnp.concatenate([W1b, pad7], axis=0),
            b1b.reshape(1, -1),
            W2a, b2a.reshape(1, -1), jnp.concatenate([W2b, pad7], axis=0),
            b2b.reshape(1, -1))

    def full(a):
        n = a.ndim
        return pl.BlockSpec(a.shape, lambda i, sc, n=n: (0,) * n)

    idx_spec = pl.BlockSpec((_BB, S), lambda i, sc: (i, 0))
    in_specs = [idx_spec, idx_spec, idx_spec] + [full(a) for a in args[3:]]

    out_shape = (
        jax.ShapeDtypeStruct((B, S, 1, 1), f32),
        jax.ShapeDtypeStruct((B, S, 1, _DIM_V), f32),
        jax.ShapeDtypeStruct((B, S, 1, _NUM_C2), f32),
    )
    out_specs = (
        pl.BlockSpec((_BB, _TC, 1, 1), lambda i, sc: (i, sc, 0, 0)),
        pl.BlockSpec((_BB, _TC, 1, _DIM_V), lambda i, sc: (i, sc, 0, 0)),
        pl.BlockSpec((_BB, _TC, 1, _NUM_C2), lambda i, sc: (i, sc, 0, 0)),
    )
    alpha4, h4, c24 = pl.pallas_call(
        _um_kernel,
        grid=(NB, S // _TC),
        in_specs=in_specs,
        out_specs=out_specs,
        out_shape=out_shape,
        scratch_shapes=[pltpu.VMEM((_BB, _DIM_V), f32),
                        pltpu.VMEM((_BB, _NUM_C2), f32),
                        pltpu.VMEM((_BB, S), f32)],
        compiler_params=pltpu.CompilerParams(
            dimension_semantics=("parallel", "arbitrary"),
            vmem_limit_bytes=128 * 1024 * 1024),
    )(*args)
    return (alpha4.reshape(B, S), h4.reshape(B, S, _DIM_V),
            c24.reshape(B, S, _NUM_C2))


# speedup vs baseline: 1.6147x; 1.0133x over previous
import jax
import jax.numpy as jnp
from jax.experimental import pallas as pl
from jax.experimental.pallas import tpu as pltpu

_NUM_C2 = 1000
_DIM_V = 64
_BB = 1024
_TC = 5


def _um_kernel(c2b, db, rb, D_t, v_d, v_c2, R_t, W_ih, W_hh, b_ih, b_hh,
               W1a, b1a, W1b, b1b, W2a, b2a, W2b, b2b,
               alpha_o, h_o, c2_o, h_st, c2_st, hist_st):
    sc = pl.program_id(1)

    @pl.when(sc == 0)
    def _():
        h_st[...] = jnp.zeros_like(h_st)
        c2_st[...] = jnp.zeros_like(c2_st)
        hist_st[...] = jnp.zeros_like(hist_st)

    c2all = c2b[...]
    dall = db[...]
    rall = rb[...]
    S = c2all.shape[1]

    iota_s = jax.lax.broadcasted_iota(jnp.int32, c2all.shape, 1)
    iota_c = jax.lax.broadcasted_iota(jnp.int32, (_BB, _NUM_C2), 1)

    def dot_t(a, b):
        return jax.lax.dot_general(a, b, (((1,), (1,)), ((), ())),
                                   preferred_element_type=jnp.float32)

    def step(tloc, _):
        t = sc * _TC + tloc
        sel_t = iota_s == t
        c2_t = jnp.sum(jnp.where(sel_t, c2all, 0), axis=1, keepdims=True)
        d_t = jnp.sum(jnp.where(sel_t, dall, 0), axis=1, keepdims=True)
        r_t = jnp.sum(jnp.where(sel_t, rall, 0), axis=1, keepdims=True)

        gamma = jnp.sum(jnp.where(iota_c == d_t, D_t[...], 0.0),
                        axis=1, keepdims=True)
        vd = gamma * v_d[...]
        vr = jnp.where(r_t == 1, R_t[1:2, :], R_t[0:1, :])

        h = h_st[...]
        x = jnp.concatenate([vd, vr], axis=1)
        gi = dot_t(x, W_ih[...]) + b_ih[...]
        gh = dot_t(h, W_hh[...]) + b_hh[...]
        r_g = jax.nn.sigmoid(gi[:, 0:64] + gh[:, 0:64])
        z_g = jax.nn.sigmoid(gi[:, 64:128] + gh[:, 64:128])
        n_g = jnp.tanh(gi[:, 128:192] + r_g * gh[:, 128:192])
        h_new = (1.0 - z_g) * n_g + z_g * h
        h_st[...] = h_new
        h_o[:, tloc, 0, :] = h_new

        t1 = jax.nn.relu(dot_t(h_new, W1a[...]) + b1a[...])
        alpha_o[:, tloc, 0, :] = dot_t(t1, W1b[...])[:, 0:1] + b1b[0, 0]

        hist = hist_st[...]
        match = jnp.logical_and(c2all == c2_t, iota_s < t)
        last = jnp.max(jnp.where(match, iota_s, -1), axis=1, keepdims=True)
        beta2 = jnp.sum(jnp.where(iota_s == last, hist, 0.0),
                        axis=1, keepdims=True)

        zcat = jnp.concatenate([beta2 * v_c2[...], vd, vr], axis=1)
        pre = jax.nn.relu(dot_t(zcat, W2a[...]) + b2a[...])
        newv = dot_t(pre, W2b[...])[:, 0:1] + b2b[0, 0]

        hist_st[...] = jnp.where(sel_t, newv, hist)
        c2n = jnp.where(iota_c == c2_t, newv, c2_st[...])
        c2_st[...] = c2n
        c2_o[:, tloc, 0, :] = c2n
        return 0

    jax.lax.fori_loop(0, _TC, step, 0)


def kernel(c1_seq, c2_seq, c4_seq, d_seq, r_seq, D_table, v_d, v_c2, R_table,
           W_ih, W_hh, b_ih, b_hh, W1a, b1a, W1b, b1b, W2a, b2a, W2b, b2b):
    del c1_seq, c4_seq
    B, S = c2_seq.shape
    NB = B // _BB
    f32 = jnp.float32

    pad7 = jnp.zeros((7, _DIM_V), f32)
    args = (c2_seq.astype(jnp.int32), d_seq.astype(jnp.int32),
            r_seq.astype(jnp.int32),
            D_table.reshape(1, -1), v_d.reshape(1, -1), v_c2.reshape(1, -1),
            R_table, W_ih, W_hh, b_ih.reshape(1, -1), b_hh.reshape(1, -1),
            W1a, b1a.reshape(1, -1), jnp.concatenate([W1b, pad7], axis=0),
            b1b.reshape(1, -1),
            W2a, b2a.reshape(1, -1), jnp.concatenate([W2b, pad7], axis=0),
            b2b.reshape(1, -1))

    def full(a):
        n = a.ndim
        return pl.BlockSpec(a.shape, lambda i, sc, n=n: (0,) * n)

    idx_spec = pl.BlockSpec((_BB, S), lambda i, sc: (i, 0))
    in_specs = [idx_spec, idx_spec, idx_spec] + [full(a) for a in args[3:]]

    out_shape = (
        jax.ShapeDtypeStruct((B, S, 1, 1), f32),
        jax.ShapeDtypeStruct((B, S, 1, _DIM_V), f32),
        jax.ShapeDtypeStruct((B, S, 1, _NUM_C2), f32),
    )
    out_specs = (
        pl.BlockSpec((_BB, _TC, 1, 1), lambda i, sc: (i, sc, 0, 0)),
        pl.BlockSpec((_BB, _TC, 1, _DIM_V), lambda i, sc: (i, sc, 0, 0)),
        pl.BlockSpec((_BB, _TC, 1, _NUM_C2), lambda i, sc: (i, sc, 0, 0)),
    )
    alpha4, h4, c24 = pl.pallas_call(
        _um_kernel,
        grid=(NB, S // _TC),
        in_specs=in_specs,
        out_specs=out_specs,
        out_shape=out_shape,
        scratch_shapes=[pltpu.VMEM((_BB, _DIM_V), f32),
                        pltpu.VMEM((_BB, _NUM_C2), f32),
                        pltpu.VMEM((_BB, S), f32)],
        compiler_params=pltpu.CompilerParams(
            dimension_semantics=("parallel", "arbitrary"),
            vmem_limit_bytes=128 * 1024 * 1024),
    )(*args)
    return (alpha4.reshape(B, S), h4.reshape(B, S, _DIM_V),
            c24.reshape(B, S, _NUM_C2))


# speedup vs baseline: 1.6157x; 1.0006x over previous
import jax
import jax.numpy as jnp
from jax.experimental import pallas as pl
from jax.experimental.pallas import tpu as pltpu

_NUM_C2 = 1000
_DIM_V = 64
_BB = 1024
_TC = 5


def _um_kernel(c2b, db, rb, D_t, v_d, v_c2, R_t, W_ih, W_hh, b_ih, b_hh,
               W1a, b1a, W1b, b1b, W2a, b2a, W2b, b2b,
               alpha_o, h_o, c2_o, h_st, c2_st, hist_st):
    sc = pl.program_id(1)

    @pl.when(sc == 0)
    def _():
        h_st[...] = jnp.zeros_like(h_st)
        c2_st[...] = jnp.zeros_like(c2_st)
        hist_st[...] = jnp.zeros_like(hist_st)

    c2all = c2b[...]
    dall = db[...]
    rall = rb[...]
    S = c2all.shape[1]

    iota_s = jax.lax.broadcasted_iota(jnp.int32, c2all.shape, 1)
    iota_c = jax.lax.broadcasted_iota(jnp.int32, (_BB, _NUM_C2), 1)

    def dot_t(a, b):
        return jax.lax.dot_general(a, b, (((1,), (1,)), ((), ())),
                                   preferred_element_type=jnp.float32)

    def step(tloc, _):
        t = sc * _TC + tloc
        sel_t = iota_s == t
        c2_t = jnp.sum(jnp.where(sel_t, c2all, 0), axis=1, keepdims=True)
        d_t = jnp.sum(jnp.where(sel_t, dall, 0), axis=1, keepdims=True)
        r_t = jnp.sum(jnp.where(sel_t, rall, 0), axis=1, keepdims=True)

        gamma = jnp.sum(jnp.where(iota_c == d_t, D_t[...], 0.0),
                        axis=1, keepdims=True)
        vd = gamma * v_d[...]
        vr = jnp.where(r_t == 1, R_t[1:2, :], R_t[0:1, :])

        h = h_st[...]
        x = jnp.concatenate([vd, vr], axis=1)
        gi = dot_t(x, W_ih[...]) + b_ih[...]
        gh = dot_t(h, W_hh[...]) + b_hh[...]
        rz = jax.nn.sigmoid(gi[:, 0:128] + gh[:, 0:128])
        r_g, z_g = rz[:, 0:64], rz[:, 64:128]
        n_g = jnp.tanh(gi[:, 128:192] + r_g * gh[:, 128:192])
        h_new = (1.0 - z_g) * n_g + z_g * h
        h_st[...] = h_new
        h_o[:, tloc, 0, :] = h_new

        t1 = jax.nn.relu(dot_t(h_new, W1a[...]) + b1a[...])
        alpha_o[:, tloc, 0, :] = dot_t(t1, W1b[...])[:, 0:1] + b1b[0, 0]

        hist = hist_st[...]
        match = jnp.logical_and(c2all == c2_t, iota_s < t)
        last = jnp.max(jnp.where(match, iota_s, -1), axis=1, keepdims=True)
        beta2 = jnp.sum(jnp.where(iota_s == last, hist, 0.0),
                        axis=1, keepdims=True)

        zcat = jnp.concatenate([beta2 * v_c2[...], vd, vr], axis=1)
        pre = jax.nn.relu(dot_t(zcat, W2a[...]) + b2a[...])
        newv = dot_t(pre, W2b[...])[:, 0:1] + b2b[0, 0]

        hist_st[...] = jnp.where(sel_t, newv, hist)
        c2n = jnp.where(iota_c == c2_t, newv, c2_st[...])
        c2_st[...] = c2n
        c2_o[:, tloc, 0, :] = c2n
        return 0

    jax.lax.fori_loop(0, _TC, step, 0)


def kernel(c1_seq, c2_seq, c4_seq, d_seq, r_seq, D_table, v_d, v_c2, R_table,
           W_ih, W_hh, b_ih, b_hh, W1a, b1a, W1b, b1b, W2a, b2a, W2b, b2b):
    del c1_seq, c4_seq
    B, S = c2_seq.shape
    NB = B // _BB
    f32 = jnp.float32

    pad7 = jnp.zeros((7, _DIM_V), f32)
    args = (c2_seq.astype(jnp.int32), d_seq.astype(jnp.int32),
            r_seq.astype(jnp.int32),
            D_table.reshape(1, -1), v_d.reshape(1, -1), v_c2.reshape(1, -1),
            R_table, W_ih, W_hh, b_ih.reshape(1, -1), b_hh.reshape(1, -1),
            W1a, b1a.reshape(1, -1), jnp.concatenate([W1b, pad7], axis=0),
            b1b.reshape(1, -1),
            W2a, b2a.reshape(1, -1), jnp.concatenate([W2b, pad7], axis=0),
            b2b.reshape(1, -1))

    def full(a):
        n = a.ndim
        return pl.BlockSpec(a.shape, lambda i, sc, n=n: (0,) * n)

    idx_spec = pl.BlockSpec((_BB, S), lambda i, sc: (i, 0))
    in_specs = [idx_spec, idx_spec, idx_spec] + [full(a) for a in args[3:]]

    out_shape = (
        jax.ShapeDtypeStruct((B, S, 1, 1), f32),
        jax.ShapeDtypeStruct((B, S, 1, _DIM_V), f32),
        jax.ShapeDtypeStruct((B, S, 1, _NUM_C2), f32),
    )
    out_specs = (
        pl.BlockSpec((_BB, _TC, 1, 1), lambda i, sc: (i, sc, 0, 0)),
        pl.BlockSpec((_BB, _TC, 1, _DIM_V), lambda i, sc: (i, sc, 0, 0)),
        pl.BlockSpec((_BB, _TC, 1, _NUM_C2), lambda i, sc: (i, sc, 0, 0)),
    )
    alpha4, h4, c24 = pl.pallas_call(
        _um_kernel,
        grid=(NB, S // _TC),
        in_specs=in_specs,
        out_specs=out_specs,
        out_shape=out_shape,
        scratch_shapes=[pltpu.VMEM((_BB, _DIM_V), f32),
                        pltpu.VMEM((_BB, _NUM_C2), f32),
                        pltpu.VMEM((_BB, S), f32)],
        compiler_params=pltpu.CompilerParams(
            dimension_semantics=("parallel", "arbitrary"),
            vmem_limit_bytes=128 * 1024 * 1024),
    )(*args)
    return (alpha4.reshape(B, S), h4.reshape(B, S, _DIM_V),
            c24.reshape(B, S, _NUM_C2))


# speedup vs baseline: 1.6627x; 1.0291x over previous
import jax
import jax.numpy as jnp
from jax.experimental import pallas as pl
from jax.experimental.pallas import tpu as pltpu

_NUM_C2 = 1000
_DIM_V = 64
_BB = 1024
_TC = 5


def _um_kernel(packedb, D_t, v_d, v_c2, R_t, W_ih, W_hh, b_ih, b_hh,
               W1a, b1a, W1b, b1b, W2a, b2a, W2b, b2b,
               alpha_o, h_o, c2_o, h_st, c2_st, hist_st):
    sc = pl.program_id(1)

    @pl.when(sc == 0)
    def _():
        h_st[...] = jnp.zeros_like(h_st)
        c2_st[...] = jnp.zeros_like(c2_st)
        hist_st[...] = jnp.zeros_like(hist_st)

    packed = packedb[...]
    c2all = jnp.bitwise_and(packed, 1023)

    iota_s = jax.lax.broadcasted_iota(jnp.int32, packed.shape, 1)
    iota_c = jax.lax.broadcasted_iota(jnp.int32, (_BB, _NUM_C2), 1)

    def dot_t(a, b):
        return jax.lax.dot_general(a, b, (((1,), (1,)), ((), ())),
                                   preferred_element_type=jnp.float32)

    def step(tloc, _):
        t = sc * _TC + tloc
        sel_t = iota_s == t
        p_t = jnp.sum(jnp.where(sel_t, packed, 0), axis=1, keepdims=True)
        c2_t = jnp.bitwise_and(p_t, 1023)
        d_t = jnp.bitwise_and(jnp.right_shift(p_t, 10), 1023)
        r_t = jnp.right_shift(p_t, 20)

        gamma = jnp.sum(jnp.where(iota_c == d_t, D_t[...], 0.0),
                        axis=1, keepdims=True)
        vd = gamma * v_d[...]
        vr = jnp.where(r_t == 1, R_t[1:2, :], R_t[0:1, :])

        h = h_st[...]
        x = jnp.concatenate([vd, vr], axis=1)
        gi = dot_t(x, W_ih[...]) + b_ih[...]
        gh = dot_t(h, W_hh[...]) + b_hh[...]
        rz = jax.nn.sigmoid(gi[:, 0:128] + gh[:, 0:128])
        r_g, z_g = rz[:, 0:64], rz[:, 64:128]
        n_g = jnp.tanh(gi[:, 128:192] + r_g * gh[:, 128:192])
        h_new = (1.0 - z_g) * n_g + z_g * h
        h_st[...] = h_new
        h_o[:, tloc, 0, :] = h_new

        t1 = jax.nn.relu(dot_t(h_new, W1a[...]) + b1a[...])
        alpha_o[:, tloc, 0, :] = dot_t(t1, W1b[...])[:, 0:1] + b1b[0, 0]

        hist = hist_st[...]
        match = jnp.logical_and(c2all == c2_t, iota_s < t)
        last = jnp.max(jnp.where(match, iota_s, -1), axis=1, keepdims=True)
        beta2 = jnp.sum(jnp.where(iota_s == last, hist, 0.0),
                        axis=1, keepdims=True)

        zcat = jnp.concatenate([beta2 * v_c2[...], vd, vr], axis=1)
        pre = jax.nn.relu(dot_t(zcat, W2a[...]) + b2a[...])
        newv = dot_t(pre, W2b[...])[:, 0:1] + b2b[0, 0]

        hist_st[...] = jnp.where(sel_t, newv, hist)
        c2n = jnp.where(iota_c == c2_t, newv, c2_st[...])
        c2_st[...] = c2n
        c2_o[:, tloc, 0, :] = c2n
        return 0

    jax.lax.fori_loop(0, _TC, step, 0)


def kernel(c1_seq, c2_seq, c4_seq, d_seq, r_seq, D_table, v_d, v_c2, R_table,
           W_ih, W_hh, b_ih, b_hh, W1a, b1a, W1b, b1b, W2a, b2a, W2b, b2b):
    del c1_seq, c4_seq
    B, S = c2_seq.shape
    NB = B // _BB
    f32 = jnp.float32

    pad7 = jnp.zeros((7, _DIM_V), f32)
    packed = (c2_seq.astype(jnp.int32)
              + (d_seq.astype(jnp.int32) << 10)
              + (r_seq.astype(jnp.int32) << 20))
    args = (packed,
            D_table.reshape(1, -1), v_d.reshape(1, -1), v_c2.reshape(1, -1),
            R_table, W_ih, W_hh, b_ih.reshape(1, -1), b_hh.reshape(1, -1),
            W1a, b1a.reshape(1, -1), jnp.concatenate([W1b, pad7], axis=0),
            b1b.reshape(1, -1),
            W2a, b2a.reshape(1, -1), jnp.concatenate([W2b, pad7], axis=0),
            b2b.reshape(1, -1))

    def full(a):
        n = a.ndim
        return pl.BlockSpec(a.shape, lambda i, sc, n=n: (0,) * n)

    idx_spec = pl.BlockSpec((_BB, S), lambda i, sc: (i, 0))
    in_specs = [idx_spec] + [full(a) for a in args[1:]]

    out_shape = (
        jax.ShapeDtypeStruct((B, S, 1, 1), f32),
        jax.ShapeDtypeStruct((B, S, 1, _DIM_V), f32),
        jax.ShapeDtypeStruct((B, S, 1, _NUM_C2), f32),
    )
    out_specs = (
        pl.BlockSpec((_BB, _TC, 1, 1), lambda i, sc: (i, sc, 0, 0)),
        pl.BlockSpec((_BB, _TC, 1, _DIM_V), lambda i, sc: (i, sc, 0, 0)),
        pl.BlockSpec((_BB, _TC, 1, _NUM_C2), lambda i, sc: (i, sc, 0, 0)),
    )
    alpha4, h4, c24 = pl.pallas_call(
        _um_kernel,
        grid=(NB, S // _TC),
        in_specs=in_specs,
        out_specs=out_specs,
        out_shape=out_shape,
        scratch_shapes=[pltpu.VMEM((_BB, _DIM_V), f32),
                        pltpu.VMEM((_BB, _NUM_C2), f32),
                        pltpu.VMEM((_BB, S), f32)],
        compiler_params=pltpu.CompilerParams(
            dimension_semantics=("parallel", "arbitrary"),
            vmem_limit_bytes=128 * 1024 * 1024),
    )(*args)
    return (alpha4.reshape(B, S), h4.reshape(B, S, _DIM_V),
            c24.reshape(B, S, _NUM_C2))
